# baseline (speedup 1.0000x reference)
import functools

import jax
import jax.numpy as jnp
from jax import lax
from jax.experimental import pallas as pl
from jax.experimental.pallas import tpu as pltpu
from jax.experimental.pallas import tpu_sc as plsc

N, E, D = 10000, 320000, 128

NW = 32
CK = 128
CH_PER_W = 80
EPW = CK * CH_PER_W
E2 = NW * EPW
PADR = 16
NR = N + PADR
RPT = 632
RPT_LAST = NR - 15 * RPT

RB = 2000
GRID = N // RB


def _dense_pre_body(x_ref, wm_ref, z_ref):
    z_ref[...] = jnp.maximum(
        jnp.dot(x_ref[...], wm_ref[...], preferred_element_type=jnp.float32),
        0.0)


_dense_pre = pl.pallas_call(
    _dense_pre_body,
    grid=(GRID,),
    in_specs=[
        pl.BlockSpec((RB, D), lambda i: (i, 0)),
        pl.BlockSpec((D, D), lambda i: (0, 0)),
    ],
    out_specs=pl.BlockSpec((RB, D), lambda i: (i, 0)),
    out_shape=jax.ShapeDtypeStruct((N, D), jnp.float32),
)


NIB = 6
NMB = 3


def _sc_edge_body(src_hbm, dst_hbm, z_hbm, zero_hbm, out_hbm,
                  sd, msg, acc, isem, gsem, ssem):
    c = lax.axis_index("c")
    s = lax.axis_index("s")
    w = c * 16 + s
    e0 = w * EPW

    def idx_load(j):
        base = pl.multiple_of(e0 + j * CK, 8)
        d1 = pltpu.async_copy(src_hbm.at[pl.ds(base, CK)], sd.at[j % NIB, 0],
                              isem[j % NIB])
        d2 = pltpu.async_copy(dst_hbm.at[pl.ds(base, CK)],
                              sd.at[j % NIB, 1], isem[j % NIB])
        return (d1, d2)

    def gather(j):
        return pltpu.async_copy(z_hbm.at[sd.at[j % NIB, 0]], msg.at[j % NMB],
                                gsem[j % NMB])

    def scatter(j):
        return pltpu.async_copy(msg.at[j % NMB], acc.at[sd.at[j % NIB, 1]],
                                ssem[j % NMB], add=True)

    iw = [idx_load(j) for j in range(4)] + [None] * (NIB - 4)
    for d in iw[0]:
        d.wait()
    gw = [gather(0)] + [None] * (NMB - 1)
    sw = [None] * NMB

    r0 = pl.multiple_of(s * RPT, 8)

    @pl.when(s < 15)
    def _():
        pltpu.sync_copy(zero_hbm.at[pl.ds(r0, RPT)], acc.at[pl.ds(r0, RPT)])

    @pl.when(s == 15)
    def _():
        pltpu.sync_copy(zero_hbm.at[pl.ds(15 * RPT, RPT_LAST)],
                        acc.at[pl.ds(15 * RPT, RPT_LAST)])

    plsc.subcore_barrier()

    def drain_scatter(b):
        if sw[b] is not None:
            sw[b].wait()
            sw[b] = None

    for j in range(CH_PER_W):
        if j + 4 < CH_PER_W:
            drain_scatter((j - 2) % NMB)
            iw[(j + 4) % NIB] = idx_load(j + 4)
        if j + 1 < CH_PER_W:
            drain_scatter((j + 1) % NMB)
            for d in iw[(j + 1) % NIB]:
                d.wait()
            gw[(j + 1) % NMB] = gather(j + 1)
        gw[j % NMB].wait()
        sw[j % NMB] = scatter(j)

    for b in range(NMB):
        drain_scatter(b)
    plsc.subcore_barrier()

    @pl.when(s < 15)
    def _():
        pltpu.sync_copy(acc.at[pl.ds(r0, RPT)], out_hbm.at[c, pl.ds(r0, RPT)])

    @pl.when(s == 15)
    def _():
        pltpu.sync_copy(acc.at[pl.ds(15 * RPT, N - 15 * RPT)],
                        out_hbm.at[c, pl.ds(15 * RPT, N - 15 * RPT)])


@functools.cache
def _sc_edge():
    return pl.kernel(
        _sc_edge_body,
        out_type=jax.ShapeDtypeStruct((2, N, D), jnp.float32),
        mesh=plsc.VectorSubcoreMesh(core_axis_name="c", subcore_axis_name="s"),
        scratch_types=[
            pltpu.VMEM((NIB, 2, CK), jnp.int32),
            pltpu.VMEM((NMB, CK, D), jnp.float32),
            pltpu.VMEM_SHARED((NR, D), jnp.float32),
            [pltpu.SemaphoreType.DMA] * NIB,
            [pltpu.SemaphoreType.DMA] * NMB,
            [pltpu.SemaphoreType.DMA] * NMB,
        ],
    )


def _epilogue_body(x_ref, p_ref, wog_ref, bo_ref, bg1_ref, wg2_ref, bg2_ref,
                   wg3_ref, bg3_ref, out_ref):
    agg = p_ref[0] + p_ref[1]
    x = x_ref[...]
    xw = jnp.dot(x, wog_ref[...], preferred_element_type=jnp.float32)
    ret = xw[:, :D] + bo_ref[...] + agg
    h = jnp.maximum(
        xw[:, D:] + bg1_ref[...]
        + jnp.dot(agg, wg2_ref[...], preferred_element_type=jnp.float32)
        + bg2_ref[...],
        0.0)
    gate = jax.nn.sigmoid(
        jnp.dot(h, wg3_ref[...], preferred_element_type=jnp.float32)
        + bg3_ref[...])
    out_ref[...] = ret * gate + x * (1.0 - gate)


_epilogue = pl.pallas_call(
    _epilogue_body,
    grid=(GRID,),
    in_specs=[
        pl.BlockSpec((RB, D), lambda i: (i, 0)),
        pl.BlockSpec((2, RB, D), lambda i: (0, i, 0)),
        pl.BlockSpec((D, 2 * D), lambda i: (0, 0)),
        pl.BlockSpec((1, D), lambda i: (0, 0)),
        pl.BlockSpec((1, D), lambda i: (0, 0)),
        pl.BlockSpec((D, D), lambda i: (0, 0)),
        pl.BlockSpec((1, D), lambda i: (0, 0)),
        pl.BlockSpec((D, D), lambda i: (0, 0)),
        pl.BlockSpec((1, D), lambda i: (0, 0)),
    ],
    out_specs=pl.BlockSpec((RB, D), lambda i: (i, 0)),
    out_shape=jax.ShapeDtypeStruct((N, D), jnp.float32),
)


def kernel(x, edge_index, Wm, W_o1, b_o1, W_g1, b_g1, W_g2, b_g2, W_g3, b_g3):
    api = jnp.arange(E2 - E, dtype=jnp.int32)
    src = jnp.concatenate([edge_index[0], api % jnp.int32(N)])
    dst = jnp.concatenate([edge_index[1], jnp.int32(N) + api % jnp.int32(PADR)])
    zeros = jnp.zeros((NR, D), jnp.float32)

    z = _dense_pre(x, Wm)
    partials = _sc_edge()(src, dst, z, zeros)
    wog = jnp.concatenate([W_o1, W_g1], axis=1)
    return _epilogue(x, partials, wog, b_o1.reshape(1, D), b_g1.reshape(1, D),
                     W_g2, b_g2.reshape(1, D), W_g3, b_g3.reshape(1, D))


# speedup vs baseline: 1.0573x; 1.0573x over previous
import functools

import jax
import jax.numpy as jnp
from jax import lax
from jax.experimental import pallas as pl
from jax.experimental.pallas import tpu as pltpu
from jax.experimental.pallas import tpu_sc as plsc

N, E, D = 10000, 320000, 128

NW = 32
CK = 128
CH_PER_W = 80
EPW = CK * CH_PER_W
E2 = NW * EPW
PADR = 16
NR = N + PADR
RPT = 632
RPT_LAST = NR - 15 * RPT

RB = 2000
GRID = N // RB


NCH = E2 // CK
NCH_REAL = E // CK
ECB = NCH // GRID


def _dense_pre_body(x_ref, wm_ref, srcb_ref, dstb_ref, z_ref, ec_ref):
    i = pl.program_id(0)
    z_ref[...] = jnp.maximum(
        jnp.dot(x_ref[...], wm_ref[...], preferred_element_type=jnp.float32),
        0.0)
    gr = lax.broadcasted_iota(jnp.int32, (ECB, CK), 0) + i * ECB
    lane = lax.broadcasted_iota(jnp.int32, (ECB, CK), 1)
    api = (gr - NCH_REAL) * CK + lane
    valid = gr < NCH_REAL
    srcv = jnp.where(valid, srcb_ref[0], api % jnp.int32(N))
    dstv = jnp.where(valid, dstb_ref[0], jnp.int32(N) + api % jnp.int32(PADR))
    ec_ref[...] = jnp.stack([srcv, dstv], axis=1)


_dense_pre = pl.pallas_call(
    _dense_pre_body,
    grid=(GRID,),
    in_specs=[
        pl.BlockSpec((RB, D), lambda i: (i, 0)),
        pl.BlockSpec((D, D), lambda i: (0, 0)),
        pl.BlockSpec((1, ECB, CK), lambda i: (0, i, 0)),
        pl.BlockSpec((1, ECB, CK), lambda i: (1, i, 0)),
    ],
    out_specs=[
        pl.BlockSpec((RB, D), lambda i: (i, 0)),
        pl.BlockSpec((ECB, 2, CK), lambda i: (i, 0, 0)),
    ],
    out_shape=[
        jax.ShapeDtypeStruct((N, D), jnp.float32),
        jax.ShapeDtypeStruct((NCH, 2, CK), jnp.int32),
    ],
)


NIB = 6
NMB = 3


def _sc_edge_body(ec_hbm, z_hbm, zero_hbm, out_hbm,
                  sd, msg, acc, isem, gsem, ssem):
    c = lax.axis_index("c")
    s = lax.axis_index("s")
    w = c * 16 + s
    ch0 = w * CH_PER_W

    def idx_load(j):
        return pltpu.async_copy(ec_hbm.at[ch0 + j], sd.at[j % NIB],
                                isem[j % NIB])

    def gather(j):
        return pltpu.async_copy(z_hbm.at[sd.at[j % NIB, 0]], msg.at[j % NMB],
                                gsem[j % NMB])

    def scatter(j):
        return pltpu.async_copy(msg.at[j % NMB], acc.at[sd.at[j % NIB, 1]],
                                ssem[j % NMB], add=True)

    iw = [idx_load(j) for j in range(4)] + [None] * (NIB - 4)
    iw[0].wait()
    gw = [gather(0)] + [None] * (NMB - 1)
    sw = [None] * NMB

    r0 = pl.multiple_of(s * RPT, 8)

    @pl.when(s < 15)
    def _():
        pltpu.sync_copy(zero_hbm.at[pl.ds(r0, RPT)], acc.at[pl.ds(r0, RPT)])

    @pl.when(s == 15)
    def _():
        pltpu.sync_copy(zero_hbm.at[pl.ds(15 * RPT, RPT_LAST)],
                        acc.at[pl.ds(15 * RPT, RPT_LAST)])

    plsc.subcore_barrier()

    def drain_scatter(b):
        if sw[b] is not None:
            sw[b].wait()
            sw[b] = None

    for j in range(CH_PER_W):
        if j + 4 < CH_PER_W:
            drain_scatter((j - 2) % NMB)
            iw[(j + 4) % NIB] = idx_load(j + 4)
        if j + 1 < CH_PER_W:
            drain_scatter((j + 1) % NMB)
            iw[(j + 1) % NIB].wait()
            gw[(j + 1) % NMB] = gather(j + 1)
        gw[j % NMB].wait()
        sw[j % NMB] = scatter(j)

    for b in range(NMB):
        drain_scatter(b)
    plsc.subcore_barrier()

    @pl.when(s < 15)
    def _():
        pltpu.sync_copy(acc.at[pl.ds(r0, RPT)], out_hbm.at[c, pl.ds(r0, RPT)])

    @pl.when(s == 15)
    def _():
        pltpu.sync_copy(acc.at[pl.ds(15 * RPT, RPT_LAST)],
                        out_hbm.at[c, pl.ds(15 * RPT, RPT_LAST)])


@functools.cache
def _sc_edge():
    return pl.kernel(
        _sc_edge_body,
        out_type=jax.ShapeDtypeStruct((2, NR, D), jnp.float32),
        mesh=plsc.VectorSubcoreMesh(core_axis_name="c", subcore_axis_name="s"),
        scratch_types=[
            pltpu.VMEM((NIB, 2, CK), jnp.int32),
            pltpu.VMEM((NMB, CK, D), jnp.float32),
            pltpu.VMEM_SHARED((NR, D), jnp.float32),
            [pltpu.SemaphoreType.DMA] * NIB,
            [pltpu.SemaphoreType.DMA] * NMB,
            [pltpu.SemaphoreType.DMA] * NMB,
        ],
    )


def _epilogue_body(x_ref, p_ref, wog_ref, bo_ref, bg1_ref, wg2_ref, bg2_ref,
                   wg3_ref, bg3_ref, out_ref):
    agg = p_ref[0] + p_ref[1]
    x = x_ref[...]
    xw = jnp.dot(x, wog_ref[...], preferred_element_type=jnp.float32)
    ret = xw[:, :D] + bo_ref[...] + agg
    h = jnp.maximum(
        xw[:, D:] + bg1_ref[...]
        + jnp.dot(agg, wg2_ref[...], preferred_element_type=jnp.float32)
        + bg2_ref[...],
        0.0)
    gate = jax.nn.sigmoid(
        jnp.dot(h, wg3_ref[...], preferred_element_type=jnp.float32)
        + bg3_ref[...])
    out_ref[...] = ret * gate + x * (1.0 - gate)


_epilogue = pl.pallas_call(
    _epilogue_body,
    grid=(GRID,),
    in_specs=[
        pl.BlockSpec((RB, D), lambda i: (i, 0)),
        pl.BlockSpec((2, RB, D), lambda i: (0, i, 0)),
        pl.BlockSpec((D, 2 * D), lambda i: (0, 0)),
        pl.BlockSpec((1, D), lambda i: (0, 0)),
        pl.BlockSpec((1, D), lambda i: (0, 0)),
        pl.BlockSpec((D, D), lambda i: (0, 0)),
        pl.BlockSpec((1, D), lambda i: (0, 0)),
        pl.BlockSpec((D, D), lambda i: (0, 0)),
        pl.BlockSpec((1, D), lambda i: (0, 0)),
    ],
    out_specs=pl.BlockSpec((RB, D), lambda i: (i, 0)),
    out_shape=jax.ShapeDtypeStruct((N, D), jnp.float32),
)


def kernel(x, edge_index, Wm, W_o1, b_o1, W_g1, b_g1, W_g2, b_g2, W_g3, b_g3):
    zeros = jnp.zeros((NR, D), jnp.float32)
    er = edge_index.reshape(2, NCH_REAL, CK)
    z, ec = _dense_pre(x, Wm, er, er)
    partials = _sc_edge()(ec, z, zeros)
    wog = jnp.concatenate([W_o1, W_g1], axis=1)
    return _epilogue(x, partials, wog, b_o1.reshape(1, D), b_g1.reshape(1, D),
                     W_g2, b_g2.reshape(1, D), W_g3, b_g3.reshape(1, D))
